# baseline (speedup 1.0000x reference)
import functools

import jax
import jax.numpy as jnp
from jax.experimental import pallas as pl
from jax.experimental.pallas import tpu as pltpu


def _greedy_search_kernel(L_ref, W_ref, sos_ref, c_ref, q_ref, *, C, T_l, J):
    W = W_ref[:]
    L = L_ref[:]

    cls_iota = jax.lax.broadcasted_iota(jnp.int32, (C, 1), 0)
    s_iota = jax.lax.broadcasted_iota(jnp.int32, (1, T_l), 1)

    def argmin_col(sim):
        m = jnp.min(sim)
        idxs = jnp.where(sim == m, cls_iota, C)
        return jnp.min(idxs)

    N = jnp.sum(L * L, axis=-1)

    q0 = jnp.tanh(jnp.dot(sos_ref[:], W, preferred_element_type=jnp.float32))
    d0 = N[:, 0:1] - 2.0 * jnp.sum(L_ref[:, 0, :] * q0, axis=-1, keepdims=True)
    c = argmin_col(d0)

    def body(t, c):
        chosen = L_ref[pl.ds(c, 1), :, :].reshape(T_l, J)
        q = jnp.tanh(jnp.dot(chosen, W, preferred_element_type=jnp.float32))
        q_ref[:] = q
        cr = jnp.sum(L * q[None, :, :], axis=-1)
        mask = (s_iota < t).astype(jnp.float32)
        sim = jnp.sum((N - 2.0 * cr) * mask, axis=-1, keepdims=True)
        return argmin_col(sim)

    c = jax.lax.fori_loop(1, T_l + 1, body, c)
    c_ref[:] = jnp.full((8, 128), c, dtype=jnp.int32)


def kernel(x, lens, W, label_seqs, sos):
    B = x.shape[0]
    C, T_l, J = label_seqs.shape

    c_tile, q = pl.pallas_call(
        functools.partial(_greedy_search_kernel, C=C, T_l=T_l, J=J),
        out_shape=(
            jax.ShapeDtypeStruct((8, 128), jnp.int32),
            jax.ShapeDtypeStruct((T_l, J), jnp.float32),
        ),
    )(label_seqs, W, sos.reshape(1, J))

    pred_label_sofar = jnp.broadcast_to(c_tile[0, 0], (B,))
    pred_label_seq = jnp.broadcast_to(q[None, :, :], (B, T_l, J))
    return (pred_label_sofar, pred_label_seq)


# speedup vs baseline: 1.5218x; 1.5218x over previous
import functools

import jax
import jax.numpy as jnp
from jax.experimental import pallas as pl
from jax.experimental.pallas import tpu as pltpu


def _greedy_search_kernel(L_ref, LT_ref, W_ref, sos_ref, c_ref, q_ref,
                          *, C, T_l, J):
    W = W_ref[:]
    LT = LT_ref[:]
    NT = jnp.sum(LT * LT, axis=1)

    lane_iota = jax.lax.broadcasted_iota(jnp.int32, (1, C), 1)
    s_iota = jax.lax.broadcasted_iota(jnp.int32, (T_l, 1), 0)

    def argmin_row(sim):
        m = jnp.min(sim)
        return jnp.min(jnp.where(sim == m, lane_iota, C))

    q0 = jnp.tanh(jnp.dot(sos_ref[:], W, preferred_element_type=jnp.float32))
    d0 = NT[0:1, :] - 2.0 * jnp.sum(LT[0] * q0.reshape(J, 1), axis=0,
                                    keepdims=True)
    c = argmin_row(d0)

    def body(t, c):
        chosen = L_ref[pl.ds(c, 1), :, :].reshape(T_l, J)
        q = jnp.tanh(jnp.dot(chosen, W, preferred_element_type=jnp.float32))
        q_ref[:] = q
        crT = jnp.sum(LT * q[:, :, None], axis=1)
        mask = (s_iota < t).astype(jnp.float32)
        sim = jnp.sum((NT - 2.0 * crT) * mask, axis=0, keepdims=True)
        return argmin_row(sim)

    c = jax.lax.fori_loop(1, T_l + 1, body, c)
    c_ref[:] = jnp.full((8, 128), c, dtype=jnp.int32)


def kernel(x, lens, W, label_seqs, sos):
    B = x.shape[0]
    C, T_l, J = label_seqs.shape

    LT = jnp.transpose(label_seqs, (1, 2, 0))

    c_tile, q = pl.pallas_call(
        functools.partial(_greedy_search_kernel, C=C, T_l=T_l, J=J),
        out_shape=(
            jax.ShapeDtypeStruct((8, 128), jnp.int32),
            jax.ShapeDtypeStruct((T_l, J), jnp.float32),
        ),
    )(label_seqs, LT, W, sos.reshape(1, J))

    pred_label_sofar = jnp.broadcast_to(c_tile[0, 0], (B,))
    pred_label_seq = jnp.broadcast_to(q[None, :, :], (B, T_l, J))
    return (pred_label_sofar, pred_label_seq)


# speedup vs baseline: 2.6434x; 1.7370x over previous
import functools

import jax
import jax.numpy as jnp
from jax.experimental import pallas as pl
from jax.experimental.pallas import tpu as pltpu


def _greedy_search_kernel(L_ref, W_ref, sos_ref, c_ref, q_ref, LT_ref,
                          *, C, T_l, J):
    W = W_ref[:]
    nt_rows = []
    for s in range(T_l):
        lt_s = jnp.transpose(L_ref[:, s, :], (1, 0))
        LT_ref[s] = lt_s
        nt_rows.append(jnp.sum(lt_s * lt_s, axis=0, keepdims=True))
    LT = LT_ref[:]
    NT = jnp.concatenate(nt_rows, axis=0)

    lane_iota = jax.lax.broadcasted_iota(jnp.int32, (1, C), 1)
    s_iota = jax.lax.broadcasted_iota(jnp.int32, (T_l, 1), 0)

    def argmin_row(sim):
        m = jnp.min(sim)
        return jnp.min(jnp.where(sim == m, lane_iota, C))

    q0 = jnp.tanh(jnp.dot(sos_ref[:], W, preferred_element_type=jnp.float32))
    d0 = NT[0:1, :] - 2.0 * jnp.sum(LT[0] * q0.reshape(J, 1), axis=0,
                                    keepdims=True)
    c = argmin_row(d0)

    def body(t, c):
        chosen = L_ref[pl.ds(c, 1), :, :].reshape(T_l, J)
        q = jnp.tanh(jnp.dot(chosen, W, preferred_element_type=jnp.float32))
        q_ref[:] = q
        crT = jnp.sum(LT * q[:, :, None], axis=1)
        mask = (s_iota < t).astype(jnp.float32)
        sim = jnp.sum((NT - 2.0 * crT) * mask, axis=0, keepdims=True)
        return argmin_row(sim)

    c = jax.lax.fori_loop(1, T_l + 1, body, c)
    c_ref[:] = jnp.full((8, 128), c, dtype=jnp.int32)


def kernel(x, lens, W, label_seqs, sos):
    B = x.shape[0]
    C, T_l, J = label_seqs.shape

    c_tile, q = pl.pallas_call(
        functools.partial(_greedy_search_kernel, C=C, T_l=T_l, J=J),
        out_shape=(
            jax.ShapeDtypeStruct((8, 128), jnp.int32),
            jax.ShapeDtypeStruct((T_l, J), jnp.float32),
        ),
        scratch_shapes=[pltpu.VMEM((T_l, J, C), jnp.float32)],
    )(label_seqs, W, sos.reshape(1, J))

    pred_label_sofar = jnp.broadcast_to(c_tile[0, 0], (B,))
    pred_label_seq = jnp.broadcast_to(q[None, :, :], (B, T_l, J))
    return (pred_label_sofar, pred_label_seq)


# speedup vs baseline: 3.6823x; 1.3930x over previous
import functools

import jax
import jax.numpy as jnp
from jax.experimental import pallas as pl
from jax.experimental.pallas import tpu as pltpu


def _greedy_search_kernel(L_ref, W_ref, sos_ref, c_ref, q_ref, LT_ref,
                          *, C, T_l, J):
    W = W_ref[:]
    nt_pre = [jnp.zeros((1, C), jnp.float32)]
    for s in range(T_l):
        lt_s = jnp.transpose(L_ref[:, s, :], (1, 0))
        LT_ref[s] = lt_s
        nt_pre.append(nt_pre[-1] + jnp.sum(lt_s * lt_s, axis=0, keepdims=True))

    lane_iota = jax.lax.broadcasted_iota(jnp.int32, (1, C), 1)

    def argmin_row(sim):
        m = jnp.min(sim)
        return jnp.min(jnp.where(sim == m, lane_iota, C))

    q0 = jnp.tanh(jnp.dot(sos_ref[:], W, preferred_element_type=jnp.float32))
    d0 = nt_pre[1] - 2.0 * jnp.sum(LT_ref[0] * q0.reshape(J, 1), axis=0,
                                   keepdims=True)
    c = argmin_row(d0)

    for t in range(1, T_l + 1):
        chosen = L_ref[pl.ds(c, 1), :, :].reshape(T_l, J)
        q = jnp.tanh(jnp.dot(chosen, W, preferred_element_type=jnp.float32))
        if t == T_l:
            q_ref[:] = q
        cross = jnp.sum(jnp.sum(LT_ref[0:t] * q[0:t, :, None], axis=1),
                        axis=0, keepdims=True)
        sim = nt_pre[t] - 2.0 * cross
        c = argmin_row(sim)

    c_ref[:] = jnp.full((8, 128), c, dtype=jnp.int32)


def kernel(x, lens, W, label_seqs, sos):
    B = x.shape[0]
    C, T_l, J = label_seqs.shape

    c_tile, q = pl.pallas_call(
        functools.partial(_greedy_search_kernel, C=C, T_l=T_l, J=J),
        out_shape=(
            jax.ShapeDtypeStruct((8, 128), jnp.int32),
            jax.ShapeDtypeStruct((T_l, J), jnp.float32),
        ),
        scratch_shapes=[pltpu.VMEM((T_l, J, C), jnp.float32)],
    )(label_seqs, W, sos.reshape(1, J))

    pred_label_sofar = jnp.broadcast_to(c_tile[0, 0], (B,))
    pred_label_seq = jnp.broadcast_to(q[None, :, :], (B, T_l, J))
    return (pred_label_sofar, pred_label_seq)


# speedup vs baseline: 4.3613x; 1.1844x over previous
import functools

import jax
import jax.numpy as jnp
from jax.experimental import pallas as pl
from jax.experimental.pallas import tpu as pltpu


def _greedy_search_kernel(L_ref, W_ref, sos_ref, c_ref, q_ref, LT_ref,
                          *, B, C, T_l, J):
    W = W_ref[:]

    nt_pre = [jnp.zeros((1, C), jnp.float32)]

    def relayout_row(s):
        lt_s = jnp.transpose(L_ref[:, s, :], (1, 0))
        LT_ref[s] = lt_s
        nt_pre.append(nt_pre[-1] + jnp.sum(lt_s * lt_s, axis=0, keepdims=True))

    lane_iota = jax.lax.broadcasted_iota(jnp.int32, (1, C), 1)

    def argmin_row(sim):
        m = jnp.min(sim)
        return jnp.min(jnp.where(sim == m, lane_iota, C))

    relayout_row(0)
    q0 = jnp.tanh(jnp.dot(sos_ref[:], W, preferred_element_type=jnp.float32))
    d0 = nt_pre[1] - 2.0 * jnp.sum(LT_ref[0] * q0.reshape(J, 1), axis=0,
                                   keepdims=True)
    c = argmin_row(d0)

    for t in range(1, T_l + 1):
        if t < T_l:
            relayout_row(t)
        chosen = L_ref[pl.ds(c, 1), :, :].reshape(T_l, J)
        q = jnp.tanh(jnp.dot(chosen, W, preferred_element_type=jnp.float32))
        if t == T_l:
            q_ref[:] = jnp.broadcast_to(q[None], (B, T_l, J))
        cross = jnp.sum(jnp.sum(LT_ref[0:t] * q[0:t, :, None], axis=1),
                        axis=0, keepdims=True)
        sim = nt_pre[t] - 2.0 * cross
        c = argmin_row(sim)

    c_ref[:] = jnp.full((1, B), c, dtype=jnp.int32)


def kernel(x, lens, W, label_seqs, sos):
    B = x.shape[0]
    C, T_l, J = label_seqs.shape

    c_row, pred_label_seq = pl.pallas_call(
        functools.partial(_greedy_search_kernel, B=B, C=C, T_l=T_l, J=J),
        out_shape=(
            jax.ShapeDtypeStruct((1, B), jnp.int32),
            jax.ShapeDtypeStruct((B, T_l, J), jnp.float32),
        ),
        scratch_shapes=[pltpu.VMEM((T_l, J, C), jnp.float32)],
    )(label_seqs, W, sos.reshape(1, J))

    return (c_row.reshape(B), pred_label_seq)
